# baseline (speedup 1.0000x reference)
import functools

import jax
import jax.numpy as jnp
import numpy as np
from jax import lax
from jax.experimental import pallas as pl
from jax.experimental.pallas import tpu as pltpu
from jax.experimental.pallas import tpu_sc as plsc

_F32 = jnp.float32


def _evenodd_perm(hdim):
    perm = np.empty((hdim,), dtype=np.int32)
    for g in range(hdim // 32):
        for j in range(16):
            perm[32 * g + j] = 32 * g + 2 * j
            perm[32 * g + 16 + j] = 32 * g + 2 * j + 1
    return perm


def _bf16_bits(x):
    u = jax.lax.bitcast_convert_type(x, jnp.uint32)
    r = u + jnp.uint32(0x7FFF) + ((u >> jnp.uint32(16)) & jnp.uint32(1))
    return r >> jnp.uint32(16)


def _pack2(acc_e, acc_o):
    w = _bf16_bits(acc_e) | (_bf16_bits(acc_o) << jnp.uint32(16))
    return jax.lax.bitcast_convert_type(w, jnp.int32)


def _mm_body(h_ref, w_ref, o_ref):
    o_ref[...] = jnp.dot(h_ref[...], w_ref[...], preferred_element_type=_F32)


def _mm(h, w):
    n, _ = h.shape
    _, hdim = w.shape
    return pl.pallas_call(
        _mm_body,
        out_shape=jax.ShapeDtypeStruct((n, hdim), _F32),
    )(h, w)


def _relu_mm_body(parts_ref, w_ref, o_ref):
    n = parts_ref.shape[0] // 2
    h = jnp.maximum(parts_ref[:n, :] + parts_ref[n:, :], 0.0)
    o_ref[...] = jnp.dot(h, w_ref[...], preferred_element_type=_F32)


def _relu_mm(parts, w):
    n = parts.shape[0] // 2
    hdim = w.shape[1]
    return pl.pallas_call(
        _relu_mm_body,
        out_shape=jax.ShapeDtypeStruct((n, hdim), _F32),
    )(parts, w)


def _filt_body(ea_ref, wf_ref, o_ref):
    acc = jnp.dot(ea_ref[...], wf_ref[...], preferred_element_type=_F32)
    o_ref[...] = pltpu.bitcast(acc.astype(jnp.bfloat16), jnp.int32)


def _filt(edge_attr, wf):
    e, de = edge_attr.shape
    hdim = wf.shape[1]
    blk = 4000
    grid = e // blk
    return pl.pallas_call(
        _filt_body,
        grid=(grid,),
        in_specs=[
            pl.BlockSpec((blk, de), lambda i: (i, 0)),
            pl.BlockSpec((de, hdim), lambda i: (0, 0)),
        ],
        out_specs=pl.BlockSpec((blk // 2, hdim), lambda i: (i, 0)),
        out_shape=jax.ShapeDtypeStruct((e // 2, hdim), jnp.int32),
    )(edge_attr, wf)


def _final_body(parts_ref, fcw_ref, fcb_ref, o_ref):
    n = parts_ref.shape[0] // 2
    h = jnp.maximum(parts_ref[:n, :] + parts_ref[n:, :], 0.0)
    pooled = jnp.mean(h, axis=0, keepdims=True)
    logits = jnp.dot(pooled, fcw_ref[...], preferred_element_type=_F32)
    logits = logits + fcb_ref[...]
    m = jnp.max(logits, axis=1, keepdims=True)
    s = logits - m
    lse = jnp.log(jnp.sum(jnp.exp(s), axis=1, keepdims=True))
    o_ref[...] = s - lse


def _final(parts, fc_w, fc_b):
    c = fc_w.shape[1]
    return pl.pallas_call(
        _final_body,
        out_shape=jax.ShapeDtypeStruct((1, c), _F32),
    )(parts, fc_w, fc_b.reshape(1, c))


@functools.lru_cache(maxsize=None)
def _make_sc_scatter(n, e, hdim):
    info = plsc.get_sparse_core_info()
    nc, ns = info.num_cores, info.num_subcores
    nw = nc * ns
    hw2 = hdim // 2
    ch = 80
    epw = e // nw
    assert epw * nw == e and epw % 8 == 0
    trips = epw // ch
    assert trips * ch == epw
    npeel = trips % 12 or 12
    assert npeel >= 2 and (trips - npeel) % 12 == 0
    cr = 40
    n_rchunks = n // cr
    assert n_rchunks * cr == n and cr <= ch
    base_r = n_rchunks // ns
    extra_r = n_rchunks - base_r * ns
    lanes = 16
    nd = 3
    ni = 4
    mesh = plsc.VectorSubcoreMesh(core_axis_name="c", subcore_axis_name="s")

    @functools.partial(
        pl.kernel,
        out_type=jax.ShapeDtypeStruct((nc * n, hdim), _F32),
        mesh=mesh,
        compiler_params=pltpu.CompilerParams(needs_layout_passes=False),
        scratch_types=[
            [pltpu.VMEM((ch,), jnp.int32) for _ in range(ni)],
            [pltpu.VMEM((ch,), jnp.int32) for _ in range(ni)],
            [pltpu.VMEM((ch, hdim), _F32) for _ in range(nd)],
            [pltpu.VMEM((ch // 2, hdim), jnp.int32) for _ in range(nd)],
            pltpu.VMEM_SHARED((n, hdim), _F32),
            [pltpu.SemaphoreType.DMA for _ in range(ni)],
            [pltpu.SemaphoreType.DMA for _ in range(nd)],
            [pltpu.SemaphoreType.DMA for _ in range(nd)],
        ],
    )
    def sc_scatter(hw_hbm, filt_hbm, src_hbm, dst_hbm, out_hbm,
                   srcs, dsts, rows, filts, acc_sp, sem_i, sem_g, sem_s):
        c = lax.axis_index("c")
        s = lax.axis_index("s")
        wid = s * nc + c

        zero_v = rows[0]

        def zfill_row(i, _):
            def zfill_col(j, _):
                zero_v[i, pl.ds(j * lanes, lanes)] = jnp.zeros((lanes,), _F32)
                return 0
            return lax.fori_loop(0, hdim // lanes, zfill_col, 0)
        lax.fori_loop(0, cr, zfill_row, 0)
        rtrips = base_r + jnp.where(s < extra_r, 1, 0)

        def zero_body(k, _):
            roff = (s + k * ns) * cr
            pltpu.sync_copy(zero_v.at[pl.ds(0, cr)],
                            acc_sp.at[pl.ds(roff, cr)])
            return 0
        lax.fori_loop(0, rtrips, zero_body, 0)
        plsc.subcore_barrier()

        base = wid * epw
        hmask = jnp.full((lanes,), -65536, jnp.int32)

        def _off(t):
            return base + jnp.minimum(t, trips - 1) * ch

        def issue_idx(t, i):
            pltpu.async_copy(src_hbm.at[pl.ds(_off(t), ch)], srcs[i],
                             sem_i[i])
            pltpu.async_copy(dst_hbm.at[pl.ds(_off(t), ch)], dsts[i],
                             sem_i[i])

        def wait_idx(i):
            pltpu.make_async_copy(src_hbm.at[pl.ds(0, ch)], srcs[i],
                                  sem_i[i]).wait()
            pltpu.make_async_copy(dst_hbm.at[pl.ds(0, ch)], dsts[i],
                                  sem_i[i]).wait()

        def _off2(t):
            off = wid * (epw // 2) + jnp.minimum(t, trips - 1) * (ch // 2)
            return pl.multiple_of(off, 8)

        def issue_gf(t, d, i):
            pltpu.async_copy(hw_hbm.at[srcs[i]], rows[d], sem_g[d])
            pltpu.async_copy(filt_hbm.at[pl.ds(_off2(t), ch // 2)],
                             filts[d], sem_g[d])

        def wait_gf(d):
            pltpu.make_async_copy(hw_hbm.at[srcs[0]], rows[d],
                                  sem_g[d]).wait()
            pltpu.make_async_copy(filt_hbm.at[pl.ds(0, ch // 2)], filts[d],
                                  sem_g[d]).wait()

        def wait_sc(d):
            pltpu.make_async_copy(rows[d], acc_sp.at[dsts[0]],
                                  sem_s[d]).wait()

        def half(t, kk, fill=False):
            d, i = kk % nd, kk % ni
            dn, i_n = (kk + 1) % nd, (kk + 1) % ni
            wait_idx(i_n)
            if not fill:
                wait_sc((kk - 2) % nd)
            issue_gf(t + 1, dn, i_n)
            wait_gf(d)
            rv, fv = rows[d], filts[d]

            @plsc.parallel_loop(0, ch // 2, unroll=4)
            def _mul_row(rr):
                for j in range(hdim // lanes):
                    sl = pl.ds(j * lanes, lanes)
                    fw = fv[rr, sl]
                    f_lo = plsc.bitcast(lax.shift_left(fw, 16), _F32)
                    f_hi = plsc.bitcast(fw & hmask, _F32)
                    rv[2 * rr, sl] = rv[2 * rr, sl] * f_lo
                    rv[2 * rr + 1, sl] = rv[2 * rr + 1, sl] * f_hi

            pltpu.async_copy(rv, acc_sp.at[dsts[i]], sem_s[d], add=True)
            issue_idx(t + 2, (kk + 2) % ni)

        issue_idx(0, 0)
        issue_idx(1, 1)
        wait_idx(0)
        issue_gf(0, 0, 0)
        for k in range(npeel):
            half(k, k % 12, fill=(k < 2))

        def body(u, _):
            t0 = npeel + u * 12
            for k in range(12):
                half(t0 + k, (npeel + k) % 12)
            return 0
        lax.fori_loop(0, (trips - npeel) // 12, body, 0)

        wait_sc((trips - 2) % nd)
        wait_sc((trips - 1) % nd)
        wait_gf(trips % nd)
        wait_idx((trips + 1) % ni)

        plsc.subcore_barrier()

        def out_body(k, _):
            roff = (s + k * ns) * cr
            pltpu.sync_copy(acc_sp.at[pl.ds(roff, cr)],
                            out_hbm.at[pl.ds(c * n + roff, cr)])
            return 0
        lax.fori_loop(0, rtrips, out_body, 0)

    return sc_scatter


def kernel(x, edge_index, edge_attr, W_0, Wf_0, W_1, Wf_1, W_2, Wf_2,
           W_3, Wf_3, fc_w, fc_b):
    n, _ = x.shape
    e = edge_attr.shape[0]
    hdim = W_0.shape[1]
    src = edge_index[0]
    dst = edge_index[1]
    sc_scatter = _make_sc_scatter(n, e, hdim)

    ws = [W_0, W_1, W_2, W_3]
    filts = [_filt(edge_attr, wf) for wf in (Wf_0, Wf_1, Wf_2, Wf_3)]
    parts = None
    for i in range(4):
        hw = _mm(x, ws[i]) if i == 0 else _relu_mm(parts, ws[i])
        parts = sc_scatter(hw, filts[i], src, dst)
    return _final(parts, fc_w, fc_b)


# speedup vs baseline: 1.0279x; 1.0279x over previous
import functools

import jax
import jax.numpy as jnp
import numpy as np
from jax import lax
from jax.experimental import pallas as pl
from jax.experimental.pallas import tpu as pltpu
from jax.experimental.pallas import tpu_sc as plsc

_F32 = jnp.float32


def _evenodd_perm(hdim):
    perm = np.empty((hdim,), dtype=np.int32)
    for g in range(hdim // 32):
        for j in range(16):
            perm[32 * g + j] = 32 * g + 2 * j
            perm[32 * g + 16 + j] = 32 * g + 2 * j + 1
    return perm


def _bf16_bits(x):
    u = jax.lax.bitcast_convert_type(x, jnp.uint32)
    r = u + jnp.uint32(0x7FFF) + ((u >> jnp.uint32(16)) & jnp.uint32(1))
    return r >> jnp.uint32(16)


def _pack2(acc_e, acc_o):
    w = _bf16_bits(acc_e) | (_bf16_bits(acc_o) << jnp.uint32(16))
    return jax.lax.bitcast_convert_type(w, jnp.int32)


def _mm_body(h_ref, w_ref, o_ref):
    o_ref[...] = jnp.dot(h_ref[...], w_ref[...], preferred_element_type=_F32)


def _mm(h, w):
    n, _ = h.shape
    _, hdim = w.shape
    return pl.pallas_call(
        _mm_body,
        out_shape=jax.ShapeDtypeStruct((n, hdim), _F32),
    )(h, w)


def _relu_mm_body(parts_ref, w_ref, o_ref):
    n = parts_ref.shape[0] // 2
    h = jnp.maximum(parts_ref[:n, :] + parts_ref[n:, :], 0.0)
    o_ref[...] = jnp.dot(h, w_ref[...], preferred_element_type=_F32)


def _relu_mm(parts, w):
    n = parts.shape[0] // 2
    hdim = w.shape[1]
    return pl.pallas_call(
        _relu_mm_body,
        out_shape=jax.ShapeDtypeStruct((n, hdim), _F32),
    )(parts, w)


def _filt_body(ea_ref, wf_ref, o_ref):
    acc = jnp.dot(ea_ref[...], wf_ref[...], preferred_element_type=_F32)
    o_ref[...] = pltpu.bitcast(acc.astype(jnp.bfloat16), jnp.int32)


def _filt(edge_attr, wf):
    e, de = edge_attr.shape
    hdim = wf.shape[1]
    blk = 4000
    grid = e // blk
    return pl.pallas_call(
        _filt_body,
        grid=(grid,),
        in_specs=[
            pl.BlockSpec((blk, de), lambda i: (i, 0)),
            pl.BlockSpec((de, hdim), lambda i: (0, 0)),
        ],
        out_specs=pl.BlockSpec((blk // 2, hdim), lambda i: (i, 0)),
        out_shape=jax.ShapeDtypeStruct((e // 2, hdim), jnp.int32),
    )(edge_attr, wf)


def _final_body(parts_ref, fcw_ref, fcb_ref, o_ref):
    n = parts_ref.shape[0] // 2
    h = jnp.maximum(parts_ref[:n, :] + parts_ref[n:, :], 0.0)
    pooled = jnp.mean(h, axis=0, keepdims=True)
    logits = jnp.dot(pooled, fcw_ref[...], preferred_element_type=_F32)
    logits = logits + fcb_ref[...]
    m = jnp.max(logits, axis=1, keepdims=True)
    s = logits - m
    lse = jnp.log(jnp.sum(jnp.exp(s), axis=1, keepdims=True))
    o_ref[...] = s - lse


def _final(parts, fc_w, fc_b):
    c = fc_w.shape[1]
    return pl.pallas_call(
        _final_body,
        out_shape=jax.ShapeDtypeStruct((1, c), _F32),
    )(parts, fc_w, fc_b.reshape(1, c))


@functools.lru_cache(maxsize=None)
def _make_sc_scatter(n, e, hdim):
    info = plsc.get_sparse_core_info()
    nc, ns = info.num_cores, info.num_subcores
    nw = nc * ns
    hw2 = hdim // 2
    ch = 80
    epw = e // nw
    assert epw * nw == e and epw % 8 == 0
    trips = epw // ch
    assert trips * ch == epw
    npeel = trips % 12 or 12
    assert npeel >= 2 and (trips - npeel) % 12 == 0
    cr = 80
    n_rchunks = n // cr
    assert n_rchunks * cr == n and cr <= ch
    base_r = n_rchunks // ns
    extra_r = n_rchunks - base_r * ns
    lanes = 16
    nd = 3
    ni = 4
    mesh = plsc.VectorSubcoreMesh(core_axis_name="c", subcore_axis_name="s")

    @functools.partial(
        pl.kernel,
        out_type=jax.ShapeDtypeStruct((nc * n, hdim), _F32),
        mesh=mesh,
        compiler_params=pltpu.CompilerParams(needs_layout_passes=False),
        scratch_types=[
            [pltpu.VMEM((ch,), jnp.int32) for _ in range(ni)],
            [pltpu.VMEM((ch,), jnp.int32) for _ in range(ni)],
            [pltpu.VMEM((ch, hdim), _F32) for _ in range(nd)],
            [pltpu.VMEM((ch // 2, hdim), jnp.int32) for _ in range(nd)],
            pltpu.VMEM_SHARED((n, hdim), _F32),
            [pltpu.SemaphoreType.DMA for _ in range(ni)],
            [pltpu.SemaphoreType.DMA for _ in range(nd)],
            [pltpu.SemaphoreType.DMA for _ in range(nd)],
        ],
    )
    def sc_scatter(hw_hbm, filt_hbm, src_hbm, dst_hbm, out_hbm,
                   srcs, dsts, rows, filts, acc_sp, sem_i, sem_g, sem_s):
        c = lax.axis_index("c")
        s = lax.axis_index("s")
        wid = s * nc + c

        zero_v = rows[0]

        def zfill_row(i, _):
            def zfill_col(j, _):
                zero_v[i, pl.ds(j * lanes, lanes)] = jnp.zeros((lanes,), _F32)
                return 0
            return lax.fori_loop(0, hdim // lanes, zfill_col, 0)
        lax.fori_loop(0, cr, zfill_row, 0)
        rtrips = base_r + jnp.where(s < extra_r, 1, 0)

        def zero_body(k, _):
            roff = (s + k * ns) * cr
            pltpu.async_copy(zero_v.at[pl.ds(0, cr)],
                             acc_sp.at[pl.ds(roff, cr)], sem_i[0])
            return 0
        lax.fori_loop(0, rtrips, zero_body, 0)

        def zero_drain(k, _):
            pltpu.make_async_copy(zero_v.at[pl.ds(0, cr)],
                                  acc_sp.at[pl.ds(0, cr)], sem_i[0]).wait()
            return 0
        lax.fori_loop(0, rtrips, zero_drain, 0)
        plsc.subcore_barrier()

        base = wid * epw
        hmask = jnp.full((lanes,), -65536, jnp.int32)

        def _off(t):
            return base + jnp.minimum(t, trips - 1) * ch

        def issue_idx(t, i):
            pltpu.async_copy(src_hbm.at[pl.ds(_off(t), ch)], srcs[i],
                             sem_i[i])
            pltpu.async_copy(dst_hbm.at[pl.ds(_off(t), ch)], dsts[i],
                             sem_i[i])

        def wait_idx(i):
            pltpu.make_async_copy(src_hbm.at[pl.ds(0, ch)], srcs[i],
                                  sem_i[i]).wait()
            pltpu.make_async_copy(dst_hbm.at[pl.ds(0, ch)], dsts[i],
                                  sem_i[i]).wait()

        def _off2(t):
            off = wid * (epw // 2) + jnp.minimum(t, trips - 1) * (ch // 2)
            return pl.multiple_of(off, 8)

        def issue_gf(t, d, i):
            pltpu.async_copy(hw_hbm.at[srcs[i]], rows[d], sem_g[d])
            pltpu.async_copy(filt_hbm.at[pl.ds(_off2(t), ch // 2)],
                             filts[d], sem_g[d])

        def wait_gf(d):
            pltpu.make_async_copy(hw_hbm.at[srcs[0]], rows[d],
                                  sem_g[d]).wait()
            pltpu.make_async_copy(filt_hbm.at[pl.ds(0, ch // 2)], filts[d],
                                  sem_g[d]).wait()

        def wait_sc(d):
            pltpu.make_async_copy(rows[d], acc_sp.at[dsts[0]],
                                  sem_s[d]).wait()

        def half(t, kk, fill=False):
            d, i = kk % nd, kk % ni
            dn, i_n = (kk + 1) % nd, (kk + 1) % ni
            wait_idx(i_n)
            if not fill:
                wait_sc((kk - 2) % nd)
            issue_gf(t + 1, dn, i_n)
            wait_gf(d)
            rv, fv = rows[d], filts[d]

            @plsc.parallel_loop(0, ch // 2, unroll=2)
            def _mul_row(rr):
                for j in range(hdim // lanes):
                    sl = pl.ds(j * lanes, lanes)
                    fw = fv[rr, sl]
                    f_lo = plsc.bitcast(lax.shift_left(fw, 16), _F32)
                    f_hi = plsc.bitcast(fw & hmask, _F32)
                    rv[2 * rr, sl] = rv[2 * rr, sl] * f_lo
                    rv[2 * rr + 1, sl] = rv[2 * rr + 1, sl] * f_hi

            pltpu.async_copy(rv, acc_sp.at[dsts[i]], sem_s[d], add=True)
            issue_idx(t + 2, (kk + 2) % ni)

        issue_idx(0, 0)
        issue_idx(1, 1)
        wait_idx(0)
        issue_gf(0, 0, 0)
        for k in range(npeel):
            half(k, k % 12, fill=(k < 2))

        def body(u, _):
            t0 = npeel + u * 12
            for k in range(12):
                half(t0 + k, (npeel + k) % 12)
            return 0
        lax.fori_loop(0, (trips - npeel) // 12, body, 0)

        wait_sc((trips - 2) % nd)
        wait_sc((trips - 1) % nd)
        wait_gf(trips % nd)
        wait_idx((trips + 1) % ni)

        plsc.subcore_barrier()

        def out_body(k, _):
            roff = (s + k * ns) * cr
            pltpu.async_copy(acc_sp.at[pl.ds(roff, cr)],
                             out_hbm.at[pl.ds(c * n + roff, cr)], sem_i[0])
            return 0
        lax.fori_loop(0, rtrips, out_body, 0)

        def out_drain(k, _):
            pltpu.make_async_copy(acc_sp.at[pl.ds(0, cr)],
                                  out_hbm.at[pl.ds(0, cr)], sem_i[0]).wait()
            return 0
        lax.fori_loop(0, rtrips, out_drain, 0)

    return sc_scatter


def kernel(x, edge_index, edge_attr, W_0, Wf_0, W_1, Wf_1, W_2, Wf_2,
           W_3, Wf_3, fc_w, fc_b):
    n, _ = x.shape
    e = edge_attr.shape[0]
    hdim = W_0.shape[1]
    src = edge_index[0]
    dst = edge_index[1]
    sc_scatter = _make_sc_scatter(n, e, hdim)

    ws = [W_0, W_1, W_2, W_3]
    filts = [_filt(edge_attr, wf) for wf in (Wf_0, Wf_1, Wf_2, Wf_3)]
    parts = None
    for i in range(4):
        hw = _mm(x, ws[i]) if i == 0 else _relu_mm(parts, ws[i])
        parts = sc_scatter(hw, filts[i], src, dst)
    return _final(parts, fc_w, fc_b)
